# baseline (speedup 1.0000x reference)
import functools

import jax
import jax.numpy as jnp
from jax import lax
from jax.experimental import pallas as pl
from jax.experimental.pallas import tpu as pltpu
from jax.experimental.pallas import tpu_sc as plsc

NC = 2
NS = 16
NW = NC * NS
L = 16
IDX_CHUNK = 128


def _build(batch, dim, n_users, n_items):
    b_per_w = batch // NW
    n_chunks = b_per_w // IDX_CHUNK
    n_groups = b_per_w // L
    mesh = plsc.VectorSubcoreMesh(
        core_axis_name="c", subcore_axis_name="s", num_cores=NC, num_subcores=NS
    )

    @functools.partial(
        pl.kernel,
        mesh=mesh,
        out_type=jax.ShapeDtypeStruct((batch,), jnp.float32),
        compiler_params=pltpu.CompilerParams(
            needs_layout_passes=False, use_tc_tiling_on_sc=False),
        scratch_types=[
            pltpu.VMEM((b_per_w, 2), jnp.int32),
            pltpu.VMEM((n_chunks, IDX_CHUNK), jnp.int32),
            pltpu.VMEM((n_chunks, IDX_CHUNK), jnp.int32),
            pltpu.VMEM((b_per_w, dim), jnp.float32),
            pltpu.VMEM((b_per_w, dim), jnp.float32),
            pltpu.VMEM((b_per_w,), jnp.float32),
            pltpu.VMEM((b_per_w,), jnp.float32),
            pltpu.VMEM((b_per_w,), jnp.float32),
            pltpu.VMEM((b_per_w,), jnp.float32),
            pltpu.SemaphoreType.DMA,
        ],
    )
    def mf(dense_hbm, sparse_hbm, p_hbm, q_hbm, ub_hbm, ib_hbm, out_hbm,
           sparse_v, uid_v, iid_v, prows_v, qrows_v, dense_v, ub_v, ib_v,
           out_v, sem):
        wid = lax.axis_index("s") * NC + lax.axis_index("c")
        base = wid * b_per_w

        pltpu.sync_copy(sparse_hbm.at[pl.ds(base, b_per_w), :], sparse_v)
        pltpu.sync_copy(dense_hbm.at[pl.ds(base, b_per_w)], dense_v)

        iota = lax.iota(jnp.int32, L)
        zeros = jnp.zeros((L,), jnp.int32)
        ones = jnp.ones((L,), jnp.int32)
        for j in range(b_per_w // L):
            rid = iota + (j * L)
            u = plsc.load_gather(sparse_v, [rid, zeros])
            t = plsc.load_gather(sparse_v, [rid, ones])
            u = jnp.where(u == 0, n_users - 1, u - 1)
            t = jnp.where(t == 0, n_items - 1, t - 1)
            row, col = divmod(j * L, IDX_CHUNK)
            uid_v[row, pl.ds(col, L)] = u
            iid_v[row, pl.ds(col, L)] = t

        copies = []
        for ck in range(n_chunks):
            r0 = ck * IDX_CHUNK
            sl = pl.ds(r0, IDX_CHUNK)
            copies.append(
                pltpu.async_copy(p_hbm.at[uid_v.at[ck]], prows_v.at[sl, :], sem))
            copies.append(
                pltpu.async_copy(q_hbm.at[iid_v.at[ck]], qrows_v.at[sl, :], sem))
            copies.append(
                pltpu.async_copy(ub_hbm.at[uid_v.at[ck]], ub_v.at[sl], sem))
            copies.append(
                pltpu.async_copy(ib_hbm.at[iid_v.at[ck]], ib_v.at[sl], sem))
        for cp in copies:
            cp.wait()

        def group(g, carry):
            rid = iota + g * L
            acc = (plsc.load_gather(dense_v, [rid])
                   + plsc.load_gather(ub_v, [rid])
                   + plsc.load_gather(ib_v, [rid]))
            for d in range(dim):
                dv = jnp.full((L,), d, jnp.int32)
                pv = plsc.load_gather(prows_v, [rid, dv])
                qv = plsc.load_gather(qrows_v, [rid, dv])
                acc = acc + pv * qv
            plsc.store_scatter(out_v, [rid], acc)
            return carry

        lax.fori_loop(0, n_groups, group, 0)
        pltpu.sync_copy(out_v, out_hbm.at[pl.ds(base, b_per_w)])

    return mf


def kernel(dense_inputs, sparse_inputs, p, q, user_bias, item_bias):
    batch = sparse_inputs.shape[0]
    mf = _build(batch, p.shape[1], p.shape[0], q.shape[0])
    out = mf(dense_inputs.reshape(-1), sparse_inputs, p, q,
             user_bias.reshape(-1), item_bias.reshape(-1))
    return out.reshape(batch, 1)
